# Initial kernel scaffold; baseline (speedup 1.0000x reference)
import jax
import jax.numpy as jnp
from jax.experimental import pallas as pl


def kernel(x, edge_index, Wres, bres, W1, as1, ad1, b1, W2, as2, ad2, b2):
    raise NotImplementedError("write your pallas kernel here")


# speedup vs baseline: 19.6161x; 19.6161x over previous
import functools

import jax
import jax.numpy as jnp
from jax import lax
from jax.experimental import pallas as pl
from jax.experimental.pallas import tpu as pltpu
from jax.experimental.pallas import tpu_sc as plsc

N = 10000
F = 128
H = 8
HID = 8
C = 64
E = 320000
EE = E + N
LANES = 16
NW = 32
RPW = 88
ROWS = NW * RPW
EP = ROWS * 128
NP = N + 16

_mesh = plsc.VectorSubcoreMesh(core_axis_name="c", subcore_axis_name="s")
_f32 = jnp.float32


@functools.partial(
    pl.kernel,
    out_type=(
        jax.ShapeDtypeStruct((ROWS, 128, LANES), _f32),
        jax.ShapeDtypeStruct((2, NP, LANES), _f32),
    ),
    mesh=_mesh,
    compiler_params=pltpu.CompilerParams(use_tc_tiling_on_sc=False),
    scratch_types=[
        pltpu.VMEM((RPW, 128), jnp.int32),
        pltpu.VMEM((RPW, 128), jnp.int32),
        pltpu.VMEM((128, LANES), _f32),
        pltpu.VMEM((128, LANES), _f32),
        pltpu.VMEM((128, LANES), _f32),
        pltpu.VMEM_SHARED((NP, LANES), _f32),
        pltpu.SemaphoreType.DMA,
        pltpu.SemaphoreType.DMA,
    ],
)
def _edge_num(srcR, dstR, stab, dtab, zeros16, ex_out, denp_out,
              srcs_v, dsts_v, g1, g2, exv, den_sh, sem1, sem2):
    c = lax.axis_index("c")
    s = lax.axis_index("s")
    wid = s * 2 + c

    @pl.when(s == 0)
    def _():
        pltpu.sync_copy(zeros16, den_sh)

    plsc.subcore_barrier()

    row0 = wid * RPW
    pltpu.sync_copy(srcR.at[pl.ds(row0, RPW)], srcs_v)
    pltpu.sync_copy(dstR.at[pl.ds(row0, RPW)], dsts_v)

    def row_body(j, carry):
        cp1 = pltpu.async_copy(stab.at[srcs_v.at[j]], g1, sem1)
        cp2 = pltpu.async_copy(dtab.at[dsts_v.at[j]], g2, sem2)
        cp1.wait()
        cp2.wait()

        def e_body(e, carry2):
            a = g1[e, :] + g2[e, :]
            a = jnp.maximum(a, 0.2 * a)
            exv[e, :] = jnp.exp(a)
            return carry2

        lax.fori_loop(0, 128, e_body, 0, unroll=4)
        pltpu.sync_copy(exv, den_sh.at[dsts_v.at[j]], add=True)
        pltpu.sync_copy(exv, ex_out.at[row0 + j])
        return carry

    lax.fori_loop(0, RPW, row_body, 0)
    plsc.subcore_barrier()

    @pl.when(s == 0)
    def _():
        pltpu.sync_copy(den_sh, denp_out.at[c])


@functools.partial(
    pl.kernel,
    out_type=(
        jax.ShapeDtypeStruct((ROWS, 128, LANES), _f32),
        jax.ShapeDtypeStruct((2, NP, 64), _f32),
    ),
    mesh=_mesh,
    compiler_params=pltpu.CompilerParams(use_tc_tiling_on_sc=False),
    scratch_types=[
        pltpu.VMEM((RPW, 128), jnp.int32),
        pltpu.VMEM((RPW, 128), jnp.int32),
        pltpu.VMEM((128, 64), _f32),
        pltpu.VMEM((128, LANES), _f32),
        pltpu.VMEM((128, LANES), _f32),
        pltpu.VMEM((128, LANES), _f32),
        pltpu.VMEM((128, 64), _f32),
        pltpu.VMEM_SHARED((NP, 64), _f32),
        pltpu.SemaphoreType.DMA,
        pltpu.SemaphoreType.DMA,
        pltpu.SemaphoreType.DMA,
    ],
)
def _edge_agg(srcR, dstR, htab, dentab, exin, zeros64, aw_out, outp_out,
              srcs_v, dsts_v, hv, denv, exv, awv, msgv, out_sh,
              semh, semd, seme):
    c = lax.axis_index("c")
    s = lax.axis_index("s")
    wid = s * 2 + c

    @pl.when(s == 0)
    def _():
        pltpu.sync_copy(zeros64, out_sh)

    plsc.subcore_barrier()

    row0 = wid * RPW
    pltpu.sync_copy(srcR.at[pl.ds(row0, RPW)], srcs_v)
    pltpu.sync_copy(dstR.at[pl.ds(row0, RPW)], dsts_v)

    def row_body(j, carry):
        cph = pltpu.async_copy(htab.at[srcs_v.at[j]], hv, semh)
        cpd = pltpu.async_copy(dentab.at[dsts_v.at[j]], denv, semd)
        cpe = pltpu.async_copy(exin.at[row0 + j], exv, seme)
        cph.wait()
        cpd.wait()
        cpe.wait()

        def e_body(e, carry2):
            alpha = exv[e, :] / denv[e, :]
            awv[e, :] = alpha
            lane = lax.iota(jnp.int32, 16)
            for jj in range(4):
                a_lo = jnp.broadcast_to(alpha[2 * jj], (16,))
                a_hi = jnp.broadcast_to(alpha[2 * jj + 1], (16,))
                rep = jnp.where(lane < HID, a_lo, a_hi)
                msgv[e, pl.ds(jj * 16, 16)] = hv[e, pl.ds(jj * 16, 16)] * rep
            return carry2

        lax.fori_loop(0, 128, e_body, 0, unroll=2)
        pltpu.sync_copy(msgv, out_sh.at[dsts_v.at[j]], add=True)
        pltpu.sync_copy(awv, aw_out.at[row0 + j])
        return carry

    lax.fori_loop(0, RPW, row_body, 0)
    plsc.subcore_barrier()

    @pl.when(s == 0)
    def _():
        pltpu.sync_copy(out_sh, outp_out.at[c])


def _dense1_body(x_ref, w1_ref, wres_ref, s1m_ref, d1m_ref,
                 h1_ref, s1_ref, d1_ref, res_ref):
    xb = x_ref[...]
    h1 = jnp.dot(xb, w1_ref[...], preferred_element_type=_f32)
    h1_ref[...] = h1
    s1_ref[...] = jnp.dot(h1, s1m_ref[...], preferred_element_type=_f32)
    d1_ref[...] = jnp.dot(h1, d1m_ref[...], preferred_element_type=_f32)
    res_ref[...] = jnp.dot(xb, wres_ref[...], preferred_element_type=_f32)


_BR = NP // 4
_dense1 = pl.pallas_call(
    _dense1_body,
    grid=(4,),
    in_specs=[
        pl.BlockSpec((_BR, F), lambda i: (i, 0)),
        pl.BlockSpec((F, 64), lambda i: (0, 0)),
        pl.BlockSpec((F, 64), lambda i: (0, 0)),
        pl.BlockSpec((64, LANES), lambda i: (0, 0)),
        pl.BlockSpec((64, LANES), lambda i: (0, 0)),
    ],
    out_specs=[
        pl.BlockSpec((_BR, 64), lambda i: (i, 0)),
        pl.BlockSpec((_BR, LANES), lambda i: (i, 0)),
        pl.BlockSpec((_BR, LANES), lambda i: (i, 0)),
        pl.BlockSpec((_BR, 64), lambda i: (i, 0)),
    ],
    out_shape=[
        jax.ShapeDtypeStruct((NP, 64), _f32),
        jax.ShapeDtypeStruct((NP, LANES), _f32),
        jax.ShapeDtypeStruct((NP, LANES), _f32),
        jax.ShapeDtypeStruct((NP, 64), _f32),
    ],
)


def _comb_body(p_ref, o_ref):
    o_ref[...] = p_ref[0] + p_ref[1] + 1e-16


_comb16 = pl.pallas_call(
    _comb_body,
    out_shape=jax.ShapeDtypeStruct((NP, LANES), _f32),
)


def _mid_body(p_ref, res_ref, b1_ref, bres_ref, w2_ref, as2m_ref, ad2m_ref,
              h2t_ref, s2b_ref, d2b_ref):
    v = p_ref[0] + p_ref[1] + b1_ref[...] + bres_ref[...] + res_ref[...]
    hmid = jnp.where(v > 0, v, jnp.exp(v) - 1.0)
    h2t = jnp.dot(hmid, w2_ref[...], preferred_element_type=_f32)
    h2t_ref[...] = h2t
    s2b_ref[...] = jnp.dot(h2t, as2m_ref[...], preferred_element_type=_f32)
    d2b_ref[...] = jnp.dot(h2t, ad2m_ref[...], preferred_element_type=_f32)


_mid = pl.pallas_call(
    _mid_body,
    out_shape=[
        jax.ShapeDtypeStruct((NP, 64), _f32),
        jax.ShapeDtypeStruct((NP, LANES), _f32),
        jax.ShapeDtypeStruct((NP, LANES), _f32),
    ],
)


def _final_body(p_ref, b2_ref, o_ref):
    v = p_ref[0] + p_ref[1] + b2_ref[...]
    z = jnp.where(v > 0, v, jnp.exp(v) - 1.0)
    m = jnp.max(z, axis=1, keepdims=True)
    lse = jnp.log(jnp.sum(jnp.exp(z - m), axis=1, keepdims=True)) + m
    o_ref[...] = z - lse


_final = pl.pallas_call(
    _final_body,
    out_shape=jax.ShapeDtypeStruct((NP, 64), _f32),
)


def kernel(x, edge_index, Wres, bres, W1, as1, ad1, b1, W2, as2, ad2, b2):
    loop = jnp.arange(N, dtype=edge_index.dtype)
    src = jnp.concatenate([edge_index[0], loop])
    dst = jnp.concatenate([edge_index[1], loop])
    ei = jnp.stack([src, dst])
    pad = jnp.full((EP - EE,), N, dtype=jnp.int32)
    srcR = jnp.concatenate([src, pad]).reshape(ROWS, 128)
    dstR = jnp.concatenate([dst, pad]).reshape(ROWS, 128)

    x_pad = jnp.pad(x, ((0, NP - N), (0, 0)))

    eye8 = jnp.eye(H, dtype=_f32)
    s1m = jnp.pad((as1[0][:, :, None] * eye8[:, None, :]).reshape(H * HID, H),
                  ((0, 0), (0, LANES - H)))
    d1m = jnp.pad((ad1[0][:, :, None] * eye8[:, None, :]).reshape(H * HID, H),
                  ((0, 0), (0, LANES - H)))
    as2m = jnp.tile(as2[0, 0][:, None], (1, LANES))
    ad2m = jnp.tile(ad2[0, 0][:, None], (1, LANES))

    z16 = jnp.zeros((NP, LANES), _f32)
    z64 = jnp.zeros((NP, 64), _f32)

    h1, s1, d1, res = _dense1(x_pad, W1, Wres, s1m, d1m)
    ex1, denp1 = _edge_num(srcR, dstR, s1, d1, z16)
    den1 = _comb16(denp1)
    aw1e, outp1 = _edge_agg(srcR, dstR, h1, den1, ex1, z64)
    h2t, s2b, d2b = _mid(outp1, res, b1.reshape(1, 64), bres.reshape(1, 64),
                         W2, as2m, ad2m)
    ex2, denp2 = _edge_num(srcR, dstR, s2b, d2b, z16)
    den2 = _comb16(denp2)
    aw2e, outp2 = _edge_agg(srcR, dstR, h2t, den2, ex2, z64)
    logp = _final(outp2, b2.reshape(1, C))[:N]

    a1 = aw1e.reshape(EP, LANES)[:EE, :H]
    a2 = aw2e.reshape(EP, LANES)[:EE, :1]
    return (logp, (ei, a1), (ei, a2))


# speedup vs baseline: 21.7449x; 1.1085x over previous
import functools

import jax
import jax.numpy as jnp
from jax import lax
from jax.experimental import pallas as pl
from jax.experimental.pallas import tpu as pltpu
from jax.experimental.pallas import tpu_sc as plsc

N = 10000
F = 128
H = 8
HID = 8
C = 64
E = 320000
EE = E + N
LANES = 16
NW = 32
RPW = 88
ROWS = NW * RPW
EP = ROWS * 128
NP = N + 16

_mesh = plsc.VectorSubcoreMesh(core_axis_name="c", subcore_axis_name="s")
_f32 = jnp.float32


@functools.partial(
    pl.kernel,
    out_type=(
        jax.ShapeDtypeStruct((ROWS, 128, LANES), _f32),
        jax.ShapeDtypeStruct((2, NP, LANES), _f32),
    ),
    mesh=_mesh,
    compiler_params=pltpu.CompilerParams(use_tc_tiling_on_sc=False),
    scratch_types=[
        pltpu.VMEM((RPW, 128), jnp.int32),
        pltpu.VMEM((RPW, 128), jnp.int32),
        pltpu.VMEM((2, 128, LANES), _f32),
        pltpu.VMEM((2, 128, LANES), _f32),
        pltpu.VMEM((2, 128, LANES), _f32),
        pltpu.VMEM_SHARED((NP, LANES), _f32),
        pltpu.SemaphoreType.DMA((2,)),
        pltpu.SemaphoreType.DMA((2,)),
        pltpu.SemaphoreType.DMA((2,)),
    ],
)
def _edge_num(srcR, dstR, stab, dtab, zeros16, ex_out, denp_out,
              srcs_v, dsts_v, g1, g2, exv, den_sh, semg, semsc, semst):
    c = lax.axis_index("c")
    s = lax.axis_index("s")
    wid = s * 2 + c

    @pl.when(s == 0)
    def _():
        pltpu.sync_copy(zeros16, den_sh)

    plsc.subcore_barrier()

    row0 = wid * RPW
    pltpu.sync_copy(srcR.at[pl.ds(row0, RPW)], srcs_v)
    pltpu.sync_copy(dstR.at[pl.ds(row0, RPW)], dsts_v)

    def issue_gather(j, b):
        pltpu.async_copy(stab.at[srcs_v.at[j]], g1.at[b], semg.at[b])
        pltpu.async_copy(dtab.at[dsts_v.at[j]], g2.at[b], semg.at[b])

    issue_gather(0, 0)

    def row_group(g, carry):
        for b in range(2):
            j = 2 * g + b

            @pl.when(j + 1 < RPW)
            def _():
                issue_gather(j + 1, 1 - b)

            pltpu.make_async_copy(stab.at[srcs_v.at[j]], g1.at[b],
                                  semg.at[b]).wait()
            pltpu.make_async_copy(dtab.at[dsts_v.at[j]], g2.at[b],
                                  semg.at[b]).wait()

            @pl.when(j >= 2)
            def _():
                pltpu.make_async_copy(exv.at[b], den_sh.at[dsts_v.at[j]],
                                      semsc.at[b]).wait()
                pltpu.make_async_copy(exv.at[b], ex_out.at[row0 + j],
                                      semst.at[b]).wait()

            def e_body(e, carry2):
                a = g1[b, e, :] + g2[b, e, :]
                a = jnp.maximum(a, 0.2 * a)
                exv[b, e, :] = jnp.exp(a)
                return carry2

            lax.fori_loop(0, 128, e_body, 0, unroll=4)
            pltpu.async_copy(exv.at[b], den_sh.at[dsts_v.at[j]],
                             semsc.at[b], add=True)
            pltpu.async_copy(exv.at[b], ex_out.at[row0 + j], semst.at[b])
        return carry

    lax.fori_loop(0, RPW // 2, row_group, 0)
    for b in range(2):
        pltpu.make_async_copy(exv.at[b], den_sh.at[dsts_v.at[0]],
                              semsc.at[b]).wait()
        pltpu.make_async_copy(exv.at[b], ex_out.at[row0], semst.at[b]).wait()
    plsc.subcore_barrier()

    @pl.when(s == 0)
    def _():
        pltpu.sync_copy(den_sh, denp_out.at[c])


@functools.partial(
    pl.kernel,
    out_type=(
        jax.ShapeDtypeStruct((ROWS, 128, LANES), _f32),
        jax.ShapeDtypeStruct((2, NP, 64), _f32),
    ),
    mesh=_mesh,
    compiler_params=pltpu.CompilerParams(use_tc_tiling_on_sc=False),
    scratch_types=[
        pltpu.VMEM((RPW, 128), jnp.int32),
        pltpu.VMEM((RPW, 128), jnp.int32),
        pltpu.VMEM((2, 128, 64), _f32),
        pltpu.VMEM((2, 128, LANES), _f32),
        pltpu.VMEM((2, 128, LANES), _f32),
        pltpu.VMEM((2, 128, LANES), _f32),
        pltpu.VMEM((2, 128, 64), _f32),
        pltpu.VMEM_SHARED((NP, 64), _f32),
        pltpu.SemaphoreType.DMA((2,)),
        pltpu.SemaphoreType.DMA((2,)),
        pltpu.SemaphoreType.DMA((2,)),
    ],
)
def _edge_agg(srcR, dstR, htab, dentab, exin, zeros64, aw_out, outp_out,
              srcs_v, dsts_v, hv, denv, exv, awv, msgv, out_sh,
              semg, semsc, semst):
    c = lax.axis_index("c")
    s = lax.axis_index("s")
    wid = s * 2 + c

    @pl.when(s == 0)
    def _():
        pltpu.sync_copy(zeros64, out_sh)

    plsc.subcore_barrier()

    row0 = wid * RPW
    pltpu.sync_copy(srcR.at[pl.ds(row0, RPW)], srcs_v)
    pltpu.sync_copy(dstR.at[pl.ds(row0, RPW)], dsts_v)

    def issue_gather(j, b):
        pltpu.async_copy(htab.at[srcs_v.at[j]], hv.at[b], semg.at[b])
        pltpu.async_copy(dentab.at[dsts_v.at[j]], denv.at[b], semg.at[b])
        pltpu.async_copy(exin.at[row0 + j], exv.at[b], semg.at[b])

    issue_gather(0, 0)

    def row_group(g, carry):
        for b in range(2):
            j = 2 * g + b

            @pl.when(j + 1 < RPW)
            def _():
                issue_gather(j + 1, 1 - b)

            pltpu.make_async_copy(htab.at[srcs_v.at[j]], hv.at[b],
                                  semg.at[b]).wait()
            pltpu.make_async_copy(dentab.at[dsts_v.at[j]], denv.at[b],
                                  semg.at[b]).wait()
            pltpu.make_async_copy(exin.at[row0 + j], exv.at[b],
                                  semg.at[b]).wait()

            @pl.when(j >= 2)
            def _():
                pltpu.make_async_copy(msgv.at[b], out_sh.at[dsts_v.at[j]],
                                      semsc.at[b]).wait()
                pltpu.make_async_copy(awv.at[b], aw_out.at[row0 + j],
                                      semst.at[b]).wait()

            def e_body(e, carry2):
                alpha = exv[b, e, :] / denv[b, e, :]
                awv[b, e, :] = alpha
                lane = lax.iota(jnp.int32, 16)
                for jj in range(4):
                    a_lo = jnp.broadcast_to(alpha[2 * jj], (16,))
                    a_hi = jnp.broadcast_to(alpha[2 * jj + 1], (16,))
                    rep = jnp.where(lane < HID, a_lo, a_hi)
                    msgv[b, e, pl.ds(jj * 16, 16)] = (
                        hv[b, e, pl.ds(jj * 16, 16)] * rep)
                return carry2

            lax.fori_loop(0, 128, e_body, 0, unroll=2)
            pltpu.async_copy(msgv.at[b], out_sh.at[dsts_v.at[j]],
                             semsc.at[b], add=True)
            pltpu.async_copy(awv.at[b], aw_out.at[row0 + j], semst.at[b])
        return carry

    lax.fori_loop(0, RPW // 2, row_group, 0)
    for b in range(2):
        pltpu.make_async_copy(msgv.at[b], out_sh.at[dsts_v.at[0]],
                              semsc.at[b]).wait()
        pltpu.make_async_copy(awv.at[b], aw_out.at[row0], semst.at[b]).wait()
    plsc.subcore_barrier()

    @pl.when(s == 0)
    def _():
        pltpu.sync_copy(out_sh, outp_out.at[c])


def _dense1_body(x_ref, w1_ref, wres_ref, s1m_ref, d1m_ref,
                 h1_ref, s1_ref, d1_ref, res_ref):
    xb = x_ref[...]
    h1 = jnp.dot(xb, w1_ref[...], preferred_element_type=_f32)
    h1_ref[...] = h1
    s1_ref[...] = jnp.dot(h1, s1m_ref[...], preferred_element_type=_f32)
    d1_ref[...] = jnp.dot(h1, d1m_ref[...], preferred_element_type=_f32)
    res_ref[...] = jnp.dot(xb, wres_ref[...], preferred_element_type=_f32)


_BR = NP // 4
_dense1 = pl.pallas_call(
    _dense1_body,
    grid=(4,),
    in_specs=[
        pl.BlockSpec((_BR, F), lambda i: (i, 0)),
        pl.BlockSpec((F, 64), lambda i: (0, 0)),
        pl.BlockSpec((F, 64), lambda i: (0, 0)),
        pl.BlockSpec((64, LANES), lambda i: (0, 0)),
        pl.BlockSpec((64, LANES), lambda i: (0, 0)),
    ],
    out_specs=[
        pl.BlockSpec((_BR, 64), lambda i: (i, 0)),
        pl.BlockSpec((_BR, LANES), lambda i: (i, 0)),
        pl.BlockSpec((_BR, LANES), lambda i: (i, 0)),
        pl.BlockSpec((_BR, 64), lambda i: (i, 0)),
    ],
    out_shape=[
        jax.ShapeDtypeStruct((NP, 64), _f32),
        jax.ShapeDtypeStruct((NP, LANES), _f32),
        jax.ShapeDtypeStruct((NP, LANES), _f32),
        jax.ShapeDtypeStruct((NP, 64), _f32),
    ],
)


def _comb_body(p_ref, o_ref):
    o_ref[...] = p_ref[0] + p_ref[1] + 1e-16


_comb16 = pl.pallas_call(
    _comb_body,
    out_shape=jax.ShapeDtypeStruct((NP, LANES), _f32),
)


def _mid_body(p_ref, res_ref, b1_ref, bres_ref, w2_ref, as2m_ref, ad2m_ref,
              h2t_ref, s2b_ref, d2b_ref):
    v = p_ref[0] + p_ref[1] + b1_ref[...] + bres_ref[...] + res_ref[...]
    hmid = jnp.where(v > 0, v, jnp.exp(v) - 1.0)
    h2t = jnp.dot(hmid, w2_ref[...], preferred_element_type=_f32)
    h2t_ref[...] = h2t
    s2b_ref[...] = jnp.dot(h2t, as2m_ref[...], preferred_element_type=_f32)
    d2b_ref[...] = jnp.dot(h2t, ad2m_ref[...], preferred_element_type=_f32)


_mid = pl.pallas_call(
    _mid_body,
    out_shape=[
        jax.ShapeDtypeStruct((NP, 64), _f32),
        jax.ShapeDtypeStruct((NP, LANES), _f32),
        jax.ShapeDtypeStruct((NP, LANES), _f32),
    ],
)


def _final_body(p_ref, b2_ref, o_ref):
    v = p_ref[0] + p_ref[1] + b2_ref[...]
    z = jnp.where(v > 0, v, jnp.exp(v) - 1.0)
    m = jnp.max(z, axis=1, keepdims=True)
    lse = jnp.log(jnp.sum(jnp.exp(z - m), axis=1, keepdims=True)) + m
    o_ref[...] = z - lse


_final = pl.pallas_call(
    _final_body,
    out_shape=jax.ShapeDtypeStruct((NP, 64), _f32),
)


def kernel(x, edge_index, Wres, bres, W1, as1, ad1, b1, W2, as2, ad2, b2):
    loop = jnp.arange(N, dtype=edge_index.dtype)
    src = jnp.concatenate([edge_index[0], loop])
    dst = jnp.concatenate([edge_index[1], loop])
    ei = jnp.stack([src, dst])
    pad = jnp.full((EP - EE,), N, dtype=jnp.int32)
    srcR = jnp.concatenate([src, pad]).reshape(ROWS, 128)
    dstR = jnp.concatenate([dst, pad]).reshape(ROWS, 128)

    x_pad = jnp.pad(x, ((0, NP - N), (0, 0)))

    eye8 = jnp.eye(H, dtype=_f32)
    s1m = jnp.pad((as1[0][:, :, None] * eye8[:, None, :]).reshape(H * HID, H),
                  ((0, 0), (0, LANES - H)))
    d1m = jnp.pad((ad1[0][:, :, None] * eye8[:, None, :]).reshape(H * HID, H),
                  ((0, 0), (0, LANES - H)))
    as2m = jnp.tile(as2[0, 0][:, None], (1, LANES))
    ad2m = jnp.tile(ad2[0, 0][:, None], (1, LANES))

    z16 = jnp.zeros((NP, LANES), _f32)
    z64 = jnp.zeros((NP, 64), _f32)

    h1, s1, d1, res = _dense1(x_pad, W1, Wres, s1m, d1m)
    ex1, denp1 = _edge_num(srcR, dstR, s1, d1, z16)
    den1 = _comb16(denp1)
    aw1e, outp1 = _edge_agg(srcR, dstR, h1, den1, ex1, z64)
    h2t, s2b, d2b = _mid(outp1, res, b1.reshape(1, 64), bres.reshape(1, 64),
                         W2, as2m, ad2m)
    ex2, denp2 = _edge_num(srcR, dstR, s2b, d2b, z16)
    den2 = _comb16(denp2)
    aw2e, outp2 = _edge_agg(srcR, dstR, h2t, den2, ex2, z64)
    logp = _final(outp2, b2.reshape(1, C))[:N]

    a1 = aw1e.reshape(EP, LANES)[:EE, :H]
    a2 = aw2e.reshape(EP, LANES)[:EE, :1]
    return (logp, (ei, a1), (ei, a2))


# speedup vs baseline: 34.4356x; 1.5836x over previous
import functools

import jax
import jax.numpy as jnp
from jax import lax
from jax.experimental import pallas as pl
from jax.experimental.pallas import tpu as pltpu
from jax.experimental.pallas import tpu_sc as plsc

N = 10000
F = 128
H = 8
HID = 8
C = 64
E = 320000
EE = E + N
LANES = 16
NW = 32
RPW = 82
ROWS = NW * RPW
EP = ROWS * 128
NP = N + 16

_mesh = plsc.VectorSubcoreMesh(core_axis_name="c", subcore_axis_name="s")
_f32 = jnp.float32


@functools.partial(
    pl.kernel,
    out_type=(
        jax.ShapeDtypeStruct((ROWS, 128, LANES), _f32),
        jax.ShapeDtypeStruct((2, NP, LANES), _f32),
    ),
    mesh=_mesh,
    compiler_params=pltpu.CompilerParams(use_tc_tiling_on_sc=False),
    scratch_types=[
        pltpu.VMEM((RPW, 128), jnp.int32),
        pltpu.VMEM((RPW, 128), jnp.int32),
        pltpu.VMEM((2, 128, LANES), _f32),
        pltpu.VMEM((2, 128, LANES), _f32),
        pltpu.VMEM((2, 128, LANES), _f32),
        pltpu.VMEM_SHARED((NP, LANES), _f32),
        pltpu.SemaphoreType.DMA((2,)),
        pltpu.SemaphoreType.DMA((2,)),
        pltpu.SemaphoreType.DMA((2,)),
    ],
)
def _edge_num(srcR, dstR, stab, dtab, zeros16, ex_out, denp_out,
              srcs_v, dsts_v, g1, g2, exv, den_sh, semg, semsc, semst):
    c = lax.axis_index("c")
    s = lax.axis_index("s")
    wid = s * 2 + c

    @pl.when(s == 0)
    def _():
        pltpu.sync_copy(zeros16, den_sh)

    plsc.subcore_barrier()

    row0 = wid * RPW
    pltpu.sync_copy(srcR.at[pl.ds(row0, RPW)], srcs_v)
    pltpu.sync_copy(dstR.at[pl.ds(row0, RPW)], dsts_v)

    def issue_gather(j, b):
        pltpu.async_copy(stab.at[srcs_v.at[j]], g1.at[b], semg.at[b])
        pltpu.async_copy(dtab.at[dsts_v.at[j]], g2.at[b], semg.at[b])

    issue_gather(0, 0)

    def row_group(g, carry):
        for b in range(2):
            j = 2 * g + b

            @pl.when(j + 1 < RPW)
            def _():
                issue_gather(j + 1, 1 - b)

            pltpu.make_async_copy(stab.at[srcs_v.at[j]], g1.at[b],
                                  semg.at[b]).wait()
            pltpu.make_async_copy(dtab.at[dsts_v.at[j]], g2.at[b],
                                  semg.at[b]).wait()

            @pl.when(j >= 2)
            def _():
                pltpu.make_async_copy(exv.at[b], den_sh.at[dsts_v.at[j]],
                                      semsc.at[b]).wait()
                pltpu.make_async_copy(exv.at[b], ex_out.at[row0 + j],
                                      semst.at[b]).wait()

            def e_body(e, carry2):
                a = g1[b, e, :] + g2[b, e, :]
                a = jnp.maximum(a, 0.2 * a)
                exv[b, e, :] = jnp.exp(a)
                return carry2

            lax.fori_loop(0, 128, e_body, 0, unroll=4)
            pltpu.async_copy(exv.at[b], den_sh.at[dsts_v.at[j]],
                             semsc.at[b], add=True)
            pltpu.async_copy(exv.at[b], ex_out.at[row0 + j], semst.at[b])
        return carry

    lax.fori_loop(0, RPW // 2, row_group, 0)
    for b in range(2):
        pltpu.make_async_copy(exv.at[b], den_sh.at[dsts_v.at[0]],
                              semsc.at[b]).wait()
        pltpu.make_async_copy(exv.at[b], ex_out.at[row0], semst.at[b]).wait()
    plsc.subcore_barrier()

    @pl.when(s == 0)
    def _():
        pltpu.sync_copy(den_sh, denp_out.at[c])


@functools.partial(
    pl.kernel,
    out_type=(
        jax.ShapeDtypeStruct((ROWS, 128, LANES), _f32),
        jax.ShapeDtypeStruct((2, NP, 64), _f32),
    ),
    mesh=_mesh,
    compiler_params=pltpu.CompilerParams(use_tc_tiling_on_sc=False),
    scratch_types=[
        pltpu.VMEM((RPW, 128), jnp.int32),
        pltpu.VMEM((RPW, 128), jnp.int32),
        pltpu.VMEM((2, 128, 64), _f32),
        pltpu.VMEM((2, 128, LANES), _f32),
        pltpu.VMEM((2, 128, LANES), _f32),
        pltpu.VMEM((2, 128, LANES), _f32),
        pltpu.VMEM((2, 128, LANES), _f32),
        pltpu.VMEM((2, 128, 64), _f32),
        pltpu.VMEM_SHARED((NP, 64), _f32),
        pltpu.SemaphoreType.DMA((2,)),
        pltpu.SemaphoreType.DMA((2,)),
        pltpu.SemaphoreType.DMA((2,)),
    ],
)
def _edge_agg(srcR, dstR, htab, den0, den1, exin, zeros64, aw_out, outp_out,
              srcs_v, dsts_v, hv, denv, denw, exv, awv, msgv, out_sh,
              semg, semsc, semst):
    c = lax.axis_index("c")
    s = lax.axis_index("s")
    wid = s * 2 + c

    @pl.when(s == 0)
    def _():
        pltpu.sync_copy(zeros64, out_sh)

    plsc.subcore_barrier()

    row0 = wid * RPW
    pltpu.sync_copy(srcR.at[pl.ds(row0, RPW)], srcs_v)
    pltpu.sync_copy(dstR.at[pl.ds(row0, RPW)], dsts_v)

    def issue_gather(j, b):
        pltpu.async_copy(htab.at[srcs_v.at[j]], hv.at[b], semg.at[b])
        pltpu.async_copy(den0.at[dsts_v.at[j]], denv.at[b], semg.at[b])
        pltpu.async_copy(den1.at[dsts_v.at[j]], denw.at[b], semg.at[b])
        pltpu.async_copy(exin.at[row0 + j], exv.at[b], semg.at[b])

    issue_gather(0, 0)

    def row_group(g, carry):
        for b in range(2):
            j = 2 * g + b

            @pl.when(j + 1 < RPW)
            def _():
                issue_gather(j + 1, 1 - b)

            pltpu.make_async_copy(htab.at[srcs_v.at[j]], hv.at[b],
                                  semg.at[b]).wait()
            pltpu.make_async_copy(den0.at[dsts_v.at[j]], denv.at[b],
                                  semg.at[b]).wait()
            pltpu.make_async_copy(den1.at[dsts_v.at[j]], denw.at[b],
                                  semg.at[b]).wait()
            pltpu.make_async_copy(exin.at[row0 + j], exv.at[b],
                                  semg.at[b]).wait()

            @pl.when(j >= 2)
            def _():
                pltpu.make_async_copy(msgv.at[b], out_sh.at[dsts_v.at[j]],
                                      semsc.at[b]).wait()
                pltpu.make_async_copy(awv.at[b], aw_out.at[row0 + j],
                                      semst.at[b]).wait()

            def e_body(e, carry2):
                alpha = exv[b, e, :] / (denv[b, e, :] + denw[b, e, :] + 1e-16)
                awv[b, e, :] = alpha
                lane = lax.iota(jnp.int32, 16)
                for jj in range(4):
                    a_lo = jnp.broadcast_to(alpha[2 * jj], (16,))
                    a_hi = jnp.broadcast_to(alpha[2 * jj + 1], (16,))
                    rep = jnp.where(lane < HID, a_lo, a_hi)
                    msgv[b, e, pl.ds(jj * 16, 16)] = (
                        hv[b, e, pl.ds(jj * 16, 16)] * rep)
                return carry2

            lax.fori_loop(0, 128, e_body, 0, unroll=2)
            pltpu.async_copy(msgv.at[b], out_sh.at[dsts_v.at[j]],
                             semsc.at[b], add=True)
            pltpu.async_copy(awv.at[b], aw_out.at[row0 + j], semst.at[b])
        return carry

    lax.fori_loop(0, RPW // 2, row_group, 0)
    for b in range(2):
        pltpu.make_async_copy(msgv.at[b], out_sh.at[dsts_v.at[0]],
                              semsc.at[b]).wait()
        pltpu.make_async_copy(awv.at[b], aw_out.at[row0], semst.at[b]).wait()
    plsc.subcore_barrier()

    @pl.when(s == 0)
    def _():
        pltpu.sync_copy(out_sh, outp_out.at[c])


def _dense1_body(x_ref, w1_ref, wres_ref, s1m_ref, d1m_ref,
                 h1_ref, s1_ref, d1_ref, res_ref):
    xb = x_ref[...]
    h1 = jnp.dot(xb, w1_ref[...], preferred_element_type=_f32)
    h1_ref[...] = h1
    s1_ref[...] = jnp.dot(h1, s1m_ref[...], preferred_element_type=_f32)
    d1_ref[...] = jnp.dot(h1, d1m_ref[...], preferred_element_type=_f32)
    res_ref[...] = jnp.dot(xb, wres_ref[...], preferred_element_type=_f32)


_BR = NP // 4
_dense1 = pl.pallas_call(
    _dense1_body,
    grid=(4,),
    in_specs=[
        pl.BlockSpec((_BR, F), lambda i: (i, 0)),
        pl.BlockSpec((F, 64), lambda i: (0, 0)),
        pl.BlockSpec((F, 64), lambda i: (0, 0)),
        pl.BlockSpec((64, LANES), lambda i: (0, 0)),
        pl.BlockSpec((64, LANES), lambda i: (0, 0)),
    ],
    out_specs=[
        pl.BlockSpec((_BR, 64), lambda i: (i, 0)),
        pl.BlockSpec((_BR, LANES), lambda i: (i, 0)),
        pl.BlockSpec((_BR, LANES), lambda i: (i, 0)),
        pl.BlockSpec((_BR, 64), lambda i: (i, 0)),
    ],
    out_shape=[
        jax.ShapeDtypeStruct((NP, 64), _f32),
        jax.ShapeDtypeStruct((NP, LANES), _f32),
        jax.ShapeDtypeStruct((NP, LANES), _f32),
        jax.ShapeDtypeStruct((NP, 64), _f32),
    ],
)


def _mid_body(p_ref, res_ref, b1_ref, bres_ref, w2_ref, as2m_ref, ad2m_ref,
              h2t_ref, s2b_ref, d2b_ref):
    v = p_ref[0] + p_ref[1] + b1_ref[...] + bres_ref[...] + res_ref[...]
    hmid = jnp.where(v > 0, v, jnp.exp(v) - 1.0)
    h2t = jnp.dot(hmid, w2_ref[...], preferred_element_type=_f32)
    h2t_ref[...] = h2t
    s2b_ref[...] = jnp.dot(h2t, as2m_ref[...], preferred_element_type=_f32)
    d2b_ref[...] = jnp.dot(h2t, ad2m_ref[...], preferred_element_type=_f32)


_mid = pl.pallas_call(
    _mid_body,
    out_shape=[
        jax.ShapeDtypeStruct((NP, 64), _f32),
        jax.ShapeDtypeStruct((NP, LANES), _f32),
        jax.ShapeDtypeStruct((NP, LANES), _f32),
    ],
)


def _final_body(p_ref, b2_ref, o_ref):
    v = p_ref[0] + p_ref[1] + b2_ref[...]
    z = jnp.where(v > 0, v, jnp.exp(v) - 1.0)
    m = jnp.max(z, axis=1, keepdims=True)
    lse = jnp.log(jnp.sum(jnp.exp(z - m), axis=1, keepdims=True)) + m
    o_ref[...] = z - lse


_final = pl.pallas_call(
    _final_body,
    out_shape=jax.ShapeDtypeStruct((NP, 64), _f32),
)


def kernel(x, edge_index, Wres, bres, W1, as1, ad1, b1, W2, as2, ad2, b2):
    loop = jnp.arange(N, dtype=edge_index.dtype)
    src = jnp.concatenate([edge_index[0], loop])
    dst = jnp.concatenate([edge_index[1], loop])
    ei = jnp.stack([src, dst])
    pad = jnp.full((EP - EE,), N, dtype=jnp.int32)
    srcR = jnp.concatenate([src, pad]).reshape(ROWS, 128)
    dstR = jnp.concatenate([dst, pad]).reshape(ROWS, 128)

    x_pad = jnp.pad(x, ((0, NP - N), (0, 0)))

    eye8 = jnp.eye(H, dtype=_f32)
    s1m = jnp.pad((as1[0][:, :, None] * eye8[:, None, :]).reshape(H * HID, H),
                  ((0, 0), (0, LANES - H)))
    d1m = jnp.pad((ad1[0][:, :, None] * eye8[:, None, :]).reshape(H * HID, H),
                  ((0, 0), (0, LANES - H)))
    as2m = jnp.tile(as2[0, 0][:, None], (1, LANES))
    ad2m = jnp.tile(ad2[0, 0][:, None], (1, LANES))

    z16 = jnp.zeros((NP, LANES), _f32)
    z64 = jnp.zeros((NP, 64), _f32)

    h1, s1, d1, res = _dense1(x_pad, W1, Wres, s1m, d1m)
    ex1, denp1 = _edge_num(srcR, dstR, s1, d1, z16)
    aw1e, outp1 = _edge_agg(srcR, dstR, h1, denp1[0], denp1[1], ex1, z64)
    h2t, s2b, d2b = _mid(outp1, res, b1.reshape(1, 64), bres.reshape(1, 64),
                         W2, as2m, ad2m)
    ex2, denp2 = _edge_num(srcR, dstR, s2b, d2b, z16)
    aw2e, outp2 = _edge_agg(srcR, dstR, h2t, denp2[0], denp2[1], ex2, z64)
    logp = _final(outp2, b2.reshape(1, C))[:N]

    a1 = aw1e.reshape(EP, LANES)[:EE, :H]
    a2 = aw2e.reshape(EP, LANES)[:EE, :1]
    return (logp, (ei, a1), (ei, a2))


# speedup vs baseline: 37.0529x; 1.0760x over previous
import functools

import jax
import jax.numpy as jnp
from jax import lax
from jax.experimental import pallas as pl
from jax.experimental.pallas import tpu as pltpu
from jax.experimental.pallas import tpu_sc as plsc

N = 10000
F = 128
H = 8
HID = 8
C = 64
E = 320000
EE = E + N
LANES = 16
NW = 32
RPW = 82
ROWS = NW * RPW
EP = ROWS * 128
NP = N + 16
FULLR = EE // 128
TAIL = EE - FULLR * 128

_mesh = plsc.VectorSubcoreMesh(core_axis_name="c", subcore_axis_name="s")
_f32 = jnp.float32


@functools.partial(
    pl.kernel,
    out_type=(
        jax.ShapeDtypeStruct((ROWS, 128, LANES), _f32),
        jax.ShapeDtypeStruct((2, NP, LANES), _f32),
    ),
    mesh=_mesh,
    compiler_params=pltpu.CompilerParams(use_tc_tiling_on_sc=False),
    scratch_types=[
        pltpu.VMEM((RPW, 128), jnp.int32),
        pltpu.VMEM((RPW, 128), jnp.int32),
        pltpu.VMEM((2, 128, LANES), _f32),
        pltpu.VMEM((2, 128, LANES), _f32),
        pltpu.VMEM((2, 128, LANES), _f32),
        pltpu.VMEM_SHARED((NP, LANES), _f32),
        pltpu.SemaphoreType.DMA((2,)),
        pltpu.SemaphoreType.DMA((2,)),
        pltpu.SemaphoreType.DMA((2,)),
    ],
)
def _edge_num(srcR, dstR, stab, dtab, zeros16, ex_out, denp_out,
              srcs_v, dsts_v, g1, g2, exv, den_sh, semg, semsc, semst):
    c = lax.axis_index("c")
    s = lax.axis_index("s")
    wid = s * 2 + c

    @pl.when(s == 0)
    def _():
        pltpu.sync_copy(zeros16, den_sh)

    plsc.subcore_barrier()

    row0 = wid * RPW
    pltpu.sync_copy(srcR.at[pl.ds(row0, RPW)], srcs_v)
    pltpu.sync_copy(dstR.at[pl.ds(row0, RPW)], dsts_v)

    def issue_gather(j, b):
        pltpu.async_copy(stab.at[srcs_v.at[j]], g1.at[b], semg.at[b])
        pltpu.async_copy(dtab.at[dsts_v.at[j]], g2.at[b], semg.at[b])

    issue_gather(0, 0)

    def row_group(g, carry):
        for b in range(2):
            j = 2 * g + b

            @pl.when(j + 1 < RPW)
            def _():
                issue_gather(j + 1, 1 - b)

            pltpu.make_async_copy(stab.at[srcs_v.at[j]], g1.at[b],
                                  semg.at[b]).wait()
            pltpu.make_async_copy(dtab.at[dsts_v.at[j]], g2.at[b],
                                  semg.at[b]).wait()

            @pl.when(j >= 2)
            def _():
                pltpu.make_async_copy(exv.at[b], den_sh.at[dsts_v.at[j]],
                                      semsc.at[b]).wait()
                pltpu.make_async_copy(exv.at[b], ex_out.at[row0 + j],
                                      semst.at[b]).wait()

            def e_body(e, carry2):
                a = g1[b, e, :] + g2[b, e, :]
                a = jnp.maximum(a, 0.2 * a)
                exv[b, e, :] = jnp.exp(a)
                return carry2

            lax.fori_loop(0, 128, e_body, 0, unroll=4)
            pltpu.async_copy(exv.at[b], den_sh.at[dsts_v.at[j]],
                             semsc.at[b], add=True)
            pltpu.async_copy(exv.at[b], ex_out.at[row0 + j], semst.at[b])
        return carry

    lax.fori_loop(0, RPW // 2, row_group, 0)
    for b in range(2):
        pltpu.make_async_copy(exv.at[b], den_sh.at[dsts_v.at[0]],
                              semsc.at[b]).wait()
        pltpu.make_async_copy(exv.at[b], ex_out.at[row0], semst.at[b]).wait()
    plsc.subcore_barrier()

    @pl.when(s == 0)
    def _():
        pltpu.sync_copy(den_sh, denp_out.at[c])


@functools.partial(
    pl.kernel,
    out_type=(
        jax.ShapeDtypeStruct((EE, HID), _f32),
        jax.ShapeDtypeStruct((2, NP, 64), _f32),
    ),
    mesh=_mesh,
    compiler_params=pltpu.CompilerParams(use_tc_tiling_on_sc=False),
    scratch_types=[
        pltpu.VMEM((RPW, 128), jnp.int32),
        pltpu.VMEM((RPW, 128), jnp.int32),
        pltpu.VMEM((2, 128, 64), _f32),
        pltpu.VMEM((2, 128, LANES), _f32),
        pltpu.VMEM((2, 128, LANES), _f32),
        pltpu.VMEM((2, 128, LANES), _f32),
        pltpu.VMEM((2, 128, LANES), _f32),
        pltpu.VMEM((2, 128, 64), _f32),
        pltpu.VMEM_SHARED((NP, 64), _f32),
        pltpu.SemaphoreType.DMA((2,)),
        pltpu.SemaphoreType.DMA((2,)),
        pltpu.SemaphoreType.DMA((2,)),
    ],
)
def _edge_agg(srcR, dstR, htab, den0, den1, exin, zeros64, aw_out, outp_out,
              srcs_v, dsts_v, hv, denv, denw, exv, awv, msgv, out_sh,
              semg, semsc, semst):
    c = lax.axis_index("c")
    s = lax.axis_index("s")
    wid = s * 2 + c

    @pl.when(s == 0)
    def _():
        pltpu.sync_copy(zeros64, out_sh)

    plsc.subcore_barrier()

    row0 = wid * RPW
    pltpu.sync_copy(srcR.at[pl.ds(row0, RPW)], srcs_v)
    pltpu.sync_copy(dstR.at[pl.ds(row0, RPW)], dsts_v)

    def issue_gather(j, b):
        pltpu.async_copy(htab.at[srcs_v.at[j]], hv.at[b], semg.at[b])
        pltpu.async_copy(den0.at[dsts_v.at[j]], denv.at[b], semg.at[b])
        pltpu.async_copy(den1.at[dsts_v.at[j]], denw.at[b], semg.at[b])
        pltpu.async_copy(exin.at[row0 + j], exv.at[b], semg.at[b])

    def aw_store(j, b):
        gr = row0 + j

        @pl.when(gr < FULLR)
        def _():
            pltpu.async_copy(awv.at[b, :, pl.ds(0, HID)],
                             aw_out.at[pl.ds(gr * 128, 128)], semst.at[b])

        @pl.when(gr == FULLR)
        def _():
            pltpu.async_copy(awv.at[b, pl.ds(0, TAIL), pl.ds(0, HID)],
                             aw_out.at[pl.ds(FULLR * 128, TAIL)], semst.at[b])

    def aw_wait(j, b):
        gr = row0 + j

        @pl.when(gr < FULLR)
        def _():
            pltpu.make_async_copy(awv.at[b, :, pl.ds(0, HID)],
                                  aw_out.at[pl.ds(0, 128)],
                                  semst.at[b]).wait()

        @pl.when(gr == FULLR)
        def _():
            pltpu.make_async_copy(awv.at[b, pl.ds(0, TAIL), pl.ds(0, HID)],
                                  aw_out.at[pl.ds(0, TAIL)],
                                  semst.at[b]).wait()

    issue_gather(0, 0)

    def row_group(g, carry):
        for b in range(2):
            j = 2 * g + b

            @pl.when(j + 1 < RPW)
            def _():
                issue_gather(j + 1, 1 - b)

            pltpu.make_async_copy(htab.at[srcs_v.at[j]], hv.at[b],
                                  semg.at[b]).wait()
            pltpu.make_async_copy(den0.at[dsts_v.at[j]], denv.at[b],
                                  semg.at[b]).wait()
            pltpu.make_async_copy(den1.at[dsts_v.at[j]], denw.at[b],
                                  semg.at[b]).wait()
            pltpu.make_async_copy(exin.at[row0 + j], exv.at[b],
                                  semg.at[b]).wait()

            @pl.when(j >= 2)
            def _():
                pltpu.make_async_copy(msgv.at[b], out_sh.at[dsts_v.at[j]],
                                      semsc.at[b]).wait()
                aw_wait(j - 2, b)

            def e_body(e, carry2):
                alpha = exv[b, e, :] / (denv[b, e, :] + denw[b, e, :] + 1e-16)
                awv[b, e, :] = alpha
                lane = lax.iota(jnp.int32, 16)
                for jj in range(4):
                    a_lo = jnp.broadcast_to(alpha[2 * jj], (16,))
                    a_hi = jnp.broadcast_to(alpha[2 * jj + 1], (16,))
                    rep = jnp.where(lane < HID, a_lo, a_hi)
                    msgv[b, e, pl.ds(jj * 16, 16)] = (
                        hv[b, e, pl.ds(jj * 16, 16)] * rep)
                return carry2

            lax.fori_loop(0, 128, e_body, 0, unroll=2)
            pltpu.async_copy(msgv.at[b], out_sh.at[dsts_v.at[j]],
                             semsc.at[b], add=True)
            aw_store(j, b)
        return carry

    lax.fori_loop(0, RPW // 2, row_group, 0)
    for b in range(2):
        pltpu.make_async_copy(msgv.at[b], out_sh.at[dsts_v.at[0]],
                              semsc.at[b]).wait()
        aw_wait(RPW - 2 + b, b)
    plsc.subcore_barrier()

    @pl.when(s == 0)
    def _():
        pltpu.sync_copy(out_sh, outp_out.at[c])


def _dense1_body(x_ref, w1_ref, wres_ref, s1m_ref, d1m_ref,
                 h1_ref, s1_ref, d1_ref, res_ref):
    xb = jnp.pad(x_ref[...], ((0, NP - N), (0, 0)))
    h1 = jnp.dot(xb, w1_ref[...], preferred_element_type=_f32)
    h1_ref[...] = h1
    s1_ref[...] = jnp.dot(h1, s1m_ref[...], preferred_element_type=_f32)
    d1_ref[...] = jnp.dot(h1, d1m_ref[...], preferred_element_type=_f32)
    res_ref[...] = jnp.dot(xb, wres_ref[...], preferred_element_type=_f32)


_dense1 = pl.pallas_call(
    _dense1_body,
    out_shape=[
        jax.ShapeDtypeStruct((NP, 64), _f32),
        jax.ShapeDtypeStruct((NP, LANES), _f32),
        jax.ShapeDtypeStruct((NP, LANES), _f32),
        jax.ShapeDtypeStruct((NP, 64), _f32),
    ],
)


def _mid_body(p_ref, res_ref, b1_ref, bres_ref, w2_ref, as2m_ref, ad2m_ref,
              h2t_ref, s2b_ref, d2b_ref):
    v = p_ref[0] + p_ref[1] + b1_ref[...] + bres_ref[...] + res_ref[...]
    hmid = jnp.where(v > 0, v, jnp.exp(v) - 1.0)
    h2t = jnp.dot(hmid, w2_ref[...], preferred_element_type=_f32)
    h2t_ref[...] = h2t
    s2b_ref[...] = jnp.dot(h2t, as2m_ref[...], preferred_element_type=_f32)
    d2b_ref[...] = jnp.dot(h2t, ad2m_ref[...], preferred_element_type=_f32)


_mid = pl.pallas_call(
    _mid_body,
    out_shape=[
        jax.ShapeDtypeStruct((NP, 64), _f32),
        jax.ShapeDtypeStruct((NP, LANES), _f32),
        jax.ShapeDtypeStruct((NP, LANES), _f32),
    ],
)


def _final_body(p_ref, b2_ref, o_ref):
    v = p_ref[0] + p_ref[1] + b2_ref[...]
    z = jnp.where(v > 0, v, jnp.exp(v) - 1.0)
    m = jnp.max(z, axis=1, keepdims=True)
    lse = jnp.log(jnp.sum(jnp.exp(z - m), axis=1, keepdims=True)) + m
    o_ref[...] = z - lse


_final = pl.pallas_call(
    _final_body,
    out_shape=jax.ShapeDtypeStruct((NP, 64), _f32),
)


def kernel(x, edge_index, Wres, bres, W1, as1, ad1, b1, W2, as2, ad2, b2):
    loop = jnp.arange(N, dtype=edge_index.dtype)
    src = jnp.concatenate([edge_index[0], loop])
    dst = jnp.concatenate([edge_index[1], loop])
    ei = jnp.stack([src, dst])
    pad = jnp.full((EP - EE,), N, dtype=jnp.int32)
    srcR = jnp.concatenate([src, pad]).reshape(ROWS, 128)
    dstR = jnp.concatenate([dst, pad]).reshape(ROWS, 128)

    eye8 = jnp.eye(H, dtype=_f32)
    s1m = jnp.pad((as1[0][:, :, None] * eye8[:, None, :]).reshape(H * HID, H),
                  ((0, 0), (0, LANES - H)))
    d1m = jnp.pad((ad1[0][:, :, None] * eye8[:, None, :]).reshape(H * HID, H),
                  ((0, 0), (0, LANES - H)))
    as2m = jnp.tile(as2[0, 0][:, None], (1, LANES))
    ad2m = jnp.tile(ad2[0, 0][:, None], (1, LANES))

    z16 = jnp.zeros((NP, LANES), _f32)
    z64 = jnp.zeros((NP, 64), _f32)

    h1, s1, d1, res = _dense1(x, W1, Wres, s1m, d1m)
    ex1, denp1 = _edge_num(srcR, dstR, s1, d1, z16)
    aw1e, outp1 = _edge_agg(srcR, dstR, h1, denp1[0], denp1[1], ex1, z64)
    h2t, s2b, d2b = _mid(outp1, res, b1.reshape(1, 64), bres.reshape(1, 64),
                         W2, as2m, ad2m)
    ex2, denp2 = _edge_num(srcR, dstR, s2b, d2b, z16)
    aw2e, outp2 = _edge_agg(srcR, dstR, h2t, denp2[0], denp2[1], ex2, z64)
    logp = _final(outp2, b2.reshape(1, C))[:N]

    return (logp, (ei, aw1e), (ei, aw2e[:, :1]))


# speedup vs baseline: 37.2459x; 1.0052x over previous
import functools

import jax
import jax.numpy as jnp
from jax import lax
from jax.experimental import pallas as pl
from jax.experimental.pallas import tpu as pltpu
from jax.experimental.pallas import tpu_sc as plsc

N = 10000
F = 128
H = 8
HID = 8
C = 64
E = 320000
EE = E + N
LANES = 16
NW = 32
RPW = 82
ROWS = NW * RPW
EP = ROWS * 128
NP = N + 16
FULLR = EE // 128
TAIL = EE - FULLR * 128

_mesh = plsc.VectorSubcoreMesh(core_axis_name="c", subcore_axis_name="s")
_f32 = jnp.float32


@functools.partial(
    pl.kernel,
    out_type=(
        jax.ShapeDtypeStruct((ROWS, 128, LANES), _f32),
        jax.ShapeDtypeStruct((2, NP, LANES), _f32),
    ),
    mesh=_mesh,
    compiler_params=pltpu.CompilerParams(use_tc_tiling_on_sc=False),
    scratch_types=[
        pltpu.VMEM((RPW, 128), jnp.int32),
        pltpu.VMEM((RPW, 128), jnp.int32),
        pltpu.VMEM((2, 128, LANES), _f32),
        pltpu.VMEM((2, 128, LANES), _f32),
        pltpu.VMEM((2, 128, LANES), _f32),
        pltpu.VMEM_SHARED((NP, LANES), _f32),
        pltpu.SemaphoreType.DMA((2,)),
        pltpu.SemaphoreType.DMA((2,)),
        pltpu.SemaphoreType.DMA((2,)),
    ],
)
def _edge_num(srcR, dstR, stab, dtab, zeros16, ex_out, denp_out,
              srcs_v, dsts_v, g1, g2, exv, den_sh, semg, semsc, semst):
    c = lax.axis_index("c")
    s = lax.axis_index("s")
    wid = s * 2 + c

    @pl.when(s == 0)
    def _():
        pltpu.sync_copy(zeros16, den_sh)

    plsc.subcore_barrier()

    row0 = wid * RPW
    pltpu.sync_copy(srcR.at[pl.ds(row0, RPW)], srcs_v)
    pltpu.sync_copy(dstR.at[pl.ds(row0, RPW)], dsts_v)

    def issue_gather(j, b):
        pltpu.async_copy(stab.at[srcs_v.at[j]], g1.at[b], semg.at[b])
        pltpu.async_copy(dtab.at[dsts_v.at[j]], g2.at[b], semg.at[b])

    issue_gather(0, 0)

    def row_group(g, carry):
        for b in range(2):
            j = 2 * g + b

            @pl.when(j + 1 < RPW)
            def _():
                issue_gather(j + 1, 1 - b)

            pltpu.make_async_copy(stab.at[srcs_v.at[j]], g1.at[b],
                                  semg.at[b]).wait()
            pltpu.make_async_copy(dtab.at[dsts_v.at[j]], g2.at[b],
                                  semg.at[b]).wait()

            @pl.when(j >= 2)
            def _():
                pltpu.make_async_copy(exv.at[b], den_sh.at[dsts_v.at[j]],
                                      semsc.at[b]).wait()
                pltpu.make_async_copy(exv.at[b], ex_out.at[row0 + j],
                                      semst.at[b]).wait()

            def e_body(e, carry2):
                a = g1[b, e, :] + g2[b, e, :]
                a = jnp.maximum(a, 0.2 * a)
                exv[b, e, :] = jnp.exp(a)
                return carry2

            lax.fori_loop(0, 128, e_body, 0, unroll=4)
            pltpu.async_copy(exv.at[b], den_sh.at[dsts_v.at[j]],
                             semsc.at[b], add=True)
            pltpu.async_copy(exv.at[b], ex_out.at[row0 + j], semst.at[b])
        return carry

    lax.fori_loop(0, RPW // 2, row_group, 0)
    for b in range(2):
        pltpu.make_async_copy(exv.at[b], den_sh.at[dsts_v.at[0]],
                              semsc.at[b]).wait()
        pltpu.make_async_copy(exv.at[b], ex_out.at[row0], semst.at[b]).wait()
    plsc.subcore_barrier()

    @pl.when(s == 0)
    def _():
        pltpu.sync_copy(den_sh, denp_out.at[c])


@functools.partial(
    pl.kernel,
    out_type=(
        jax.ShapeDtypeStruct((EE, HID), _f32),
        jax.ShapeDtypeStruct((2, NP, 64), _f32),
    ),
    mesh=_mesh,
    compiler_params=pltpu.CompilerParams(use_tc_tiling_on_sc=False),
    scratch_types=[
        pltpu.VMEM((RPW, 128), jnp.int32),
        pltpu.VMEM((RPW, 128), jnp.int32),
        pltpu.VMEM((2, 128, 64), _f32),
        pltpu.VMEM((2, 128, LANES), _f32),
        pltpu.VMEM((2, 128, LANES), _f32),
        pltpu.VMEM((2, 128, LANES), _f32),
        pltpu.VMEM((2, 128, LANES), _f32),
        pltpu.VMEM((2, 128, 64), _f32),
        pltpu.VMEM_SHARED((NP, 64), _f32),
        pltpu.SemaphoreType.DMA((2,)),
        pltpu.SemaphoreType.DMA((2,)),
        pltpu.SemaphoreType.DMA((2,)),
    ],
)
def _edge_agg(srcR, dstR, htab, den0, den1, exin, zeros64, aw_out, outp_out,
              srcs_v, dsts_v, hv, denv, denw, exv, awv, msgv, out_sh,
              semg, semsc, semst):
    c = lax.axis_index("c")
    s = lax.axis_index("s")
    wid = s * 2 + c

    @pl.when(s == 0)
    def _():
        pltpu.sync_copy(zeros64, out_sh)

    plsc.subcore_barrier()

    row0 = wid * RPW
    pltpu.sync_copy(srcR.at[pl.ds(row0, RPW)], srcs_v)
    pltpu.sync_copy(dstR.at[pl.ds(row0, RPW)], dsts_v)

    def issue_gather(j, b):
        pltpu.async_copy(htab.at[srcs_v.at[j]], hv.at[b], semg.at[b])
        pltpu.async_copy(den0.at[dsts_v.at[j]], denv.at[b], semg.at[b])
        pltpu.async_copy(den1.at[dsts_v.at[j]], denw.at[b], semg.at[b])
        pltpu.async_copy(exin.at[row0 + j], exv.at[b], semg.at[b])

    def aw_store(j, b):
        gr = row0 + j

        @pl.when(gr < FULLR)
        def _():
            pltpu.async_copy(awv.at[b, :, pl.ds(0, HID)],
                             aw_out.at[pl.ds(gr * 128, 128)], semst.at[b])

        @pl.when(gr == FULLR)
        def _():
            pltpu.async_copy(awv.at[b, pl.ds(0, TAIL), pl.ds(0, HID)],
                             aw_out.at[pl.ds(FULLR * 128, TAIL)], semst.at[b])

    def aw_wait(j, b):
        gr = row0 + j

        @pl.when(gr < FULLR)
        def _():
            pltpu.make_async_copy(awv.at[b, :, pl.ds(0, HID)],
                                  aw_out.at[pl.ds(0, 128)],
                                  semst.at[b]).wait()

        @pl.when(gr == FULLR)
        def _():
            pltpu.make_async_copy(awv.at[b, pl.ds(0, TAIL), pl.ds(0, HID)],
                                  aw_out.at[pl.ds(0, TAIL)],
                                  semst.at[b]).wait()

    issue_gather(0, 0)

    def row_group(g, carry):
        for b in range(2):
            j = 2 * g + b

            @pl.when(j + 1 < RPW)
            def _():
                issue_gather(j + 1, 1 - b)

            pltpu.make_async_copy(htab.at[srcs_v.at[j]], hv.at[b],
                                  semg.at[b]).wait()
            pltpu.make_async_copy(den0.at[dsts_v.at[j]], denv.at[b],
                                  semg.at[b]).wait()
            pltpu.make_async_copy(den1.at[dsts_v.at[j]], denw.at[b],
                                  semg.at[b]).wait()
            pltpu.make_async_copy(exin.at[row0 + j], exv.at[b],
                                  semg.at[b]).wait()

            @pl.when(j >= 2)
            def _():
                pltpu.make_async_copy(msgv.at[b], out_sh.at[dsts_v.at[j]],
                                      semsc.at[b]).wait()
                aw_wait(j - 2, b)

            lomask = lax.iota(jnp.int32, 16) < HID

            def e_body(e, carry2):
                alpha = exv[b, e, :] / (denv[b, e, :] + denw[b, e, :] + 1e-16)
                awv[b, e, :] = alpha
                for jj in range(4):
                    a_lo = jnp.broadcast_to(alpha[2 * jj], (16,))
                    a_hi = jnp.broadcast_to(alpha[2 * jj + 1], (16,))
                    rep = jnp.where(lomask, a_lo, a_hi)
                    msgv[b, e, pl.ds(jj * 16, 16)] = (
                        hv[b, e, pl.ds(jj * 16, 16)] * rep)
                return carry2

            lax.fori_loop(0, 128, e_body, 0, unroll=4)
            pltpu.async_copy(msgv.at[b], out_sh.at[dsts_v.at[j]],
                             semsc.at[b], add=True)
            aw_store(j, b)
        return carry

    lax.fori_loop(0, RPW // 2, row_group, 0)
    for b in range(2):
        pltpu.make_async_copy(msgv.at[b], out_sh.at[dsts_v.at[0]],
                              semsc.at[b]).wait()
        aw_wait(RPW - 2 + b, b)
    plsc.subcore_barrier()

    @pl.when(s == 0)
    def _():
        pltpu.sync_copy(out_sh, outp_out.at[c])


def _dense1_body(x_ref, w1_ref, wres_ref, s1m_ref, d1m_ref,
                 h1_ref, s1_ref, d1_ref, res_ref):
    xb = jnp.pad(x_ref[...], ((0, NP - N), (0, 0)))
    h1 = jnp.dot(xb, w1_ref[...], preferred_element_type=_f32)
    h1_ref[...] = h1
    s1_ref[...] = jnp.dot(h1, s1m_ref[...], preferred_element_type=_f32)
    d1_ref[...] = jnp.dot(h1, d1m_ref[...], preferred_element_type=_f32)
    res_ref[...] = jnp.dot(xb, wres_ref[...], preferred_element_type=_f32)


_dense1 = pl.pallas_call(
    _dense1_body,
    out_shape=[
        jax.ShapeDtypeStruct((NP, 64), _f32),
        jax.ShapeDtypeStruct((NP, LANES), _f32),
        jax.ShapeDtypeStruct((NP, LANES), _f32),
        jax.ShapeDtypeStruct((NP, 64), _f32),
    ],
)


def _mid_body(p_ref, res_ref, b1_ref, bres_ref, w2_ref, as2m_ref, ad2m_ref,
              h2t_ref, s2b_ref, d2b_ref):
    v = p_ref[0] + p_ref[1] + b1_ref[...] + bres_ref[...] + res_ref[...]
    hmid = jnp.where(v > 0, v, jnp.exp(v) - 1.0)
    h2t = jnp.dot(hmid, w2_ref[...], preferred_element_type=_f32)
    h2t_ref[...] = h2t
    s2b_ref[...] = jnp.dot(h2t, as2m_ref[...], preferred_element_type=_f32)
    d2b_ref[...] = jnp.dot(h2t, ad2m_ref[...], preferred_element_type=_f32)


_mid = pl.pallas_call(
    _mid_body,
    out_shape=[
        jax.ShapeDtypeStruct((NP, 64), _f32),
        jax.ShapeDtypeStruct((NP, LANES), _f32),
        jax.ShapeDtypeStruct((NP, LANES), _f32),
    ],
)


def _final_body(p_ref, b2_ref, o_ref):
    v = p_ref[0, :N] + p_ref[1, :N] + b2_ref[...]
    z = jnp.where(v > 0, v, jnp.exp(v) - 1.0)
    m = jnp.max(z, axis=1, keepdims=True)
    lse = jnp.log(jnp.sum(jnp.exp(z - m), axis=1, keepdims=True)) + m
    o_ref[...] = z - lse


_final = pl.pallas_call(
    _final_body,
    out_shape=jax.ShapeDtypeStruct((N, 64), _f32),
)


def kernel(x, edge_index, Wres, bres, W1, as1, ad1, b1, W2, as2, ad2, b2):
    loop = jnp.arange(N, dtype=edge_index.dtype)
    src = jnp.concatenate([edge_index[0], loop])
    dst = jnp.concatenate([edge_index[1], loop])
    ei = jnp.stack([src, dst])
    pad = jnp.full((EP - EE,), N, dtype=jnp.int32)
    srcR = jnp.concatenate([src, pad]).reshape(ROWS, 128)
    dstR = jnp.concatenate([dst, pad]).reshape(ROWS, 128)

    eye8 = jnp.eye(H, dtype=_f32)
    s1m = jnp.pad((as1[0][:, :, None] * eye8[:, None, :]).reshape(H * HID, H),
                  ((0, 0), (0, LANES - H)))
    d1m = jnp.pad((ad1[0][:, :, None] * eye8[:, None, :]).reshape(H * HID, H),
                  ((0, 0), (0, LANES - H)))
    as2m = jnp.tile(as2[0, 0][:, None], (1, LANES))
    ad2m = jnp.tile(ad2[0, 0][:, None], (1, LANES))

    z16 = jnp.zeros((NP, LANES), _f32)
    z64 = jnp.zeros((NP, 64), _f32)

    h1, s1, d1, res = _dense1(x, W1, Wres, s1m, d1m)
    ex1, denp1 = _edge_num(srcR, dstR, s1, d1, z16)
    aw1e, outp1 = _edge_agg(srcR, dstR, h1, denp1[0], denp1[1], ex1, z64)
    h2t, s2b, d2b = _mid(outp1, res, b1.reshape(1, 64), bres.reshape(1, 64),
                         W2, as2m, ad2m)
    ex2, denp2 = _edge_num(srcR, dstR, s2b, d2b, z16)
    aw2e, outp2 = _edge_agg(srcR, dstR, h2t, denp2[0], denp2[1], ex2, z64)
    logp = _final(outp2, b2.reshape(1, C))

    return (logp, (ei, aw1e), (ei, aw2e[:, :1]))


# speedup vs baseline: 37.9579x; 1.0191x over previous
import functools

import jax
import jax.numpy as jnp
from jax import lax
from jax.experimental import pallas as pl
from jax.experimental.pallas import tpu as pltpu
from jax.experimental.pallas import tpu_sc as plsc

N = 10000
F = 128
H = 8
HID = 8
C = 64
E = 320000
EE = E + N
LANES = 16
NW = 32
RPW = 82
ROWS = NW * RPW
EP = ROWS * 128
NP = N + 16
FULLR = EE // 128
TAIL = EE - FULLR * 128

_mesh = plsc.VectorSubcoreMesh(core_axis_name="c", subcore_axis_name="s")
_f32 = jnp.float32


@functools.partial(
    pl.kernel,
    out_type=(
        jax.ShapeDtypeStruct((ROWS, 128, LANES), _f32),
        jax.ShapeDtypeStruct((2, NP, LANES), _f32),
    ),
    mesh=_mesh,
    compiler_params=pltpu.CompilerParams(use_tc_tiling_on_sc=False),
    scratch_types=[
        pltpu.VMEM((RPW, 128), jnp.int32),
        pltpu.VMEM((RPW, 128), jnp.int32),
        pltpu.VMEM((2, 128, LANES), _f32),
        pltpu.VMEM((2, 128, LANES), _f32),
        pltpu.VMEM((2, 128, LANES), _f32),
        pltpu.VMEM_SHARED((NP, LANES), _f32),
        pltpu.SemaphoreType.DMA((2,)),
        pltpu.SemaphoreType.DMA((2,)),
        pltpu.SemaphoreType.DMA((2,)),
    ],
)
def _edge_num(srcR, dstR, stab, dtab, zeros16, ex_out, denp_out,
              srcs_v, dsts_v, g1, g2, exv, den_sh, semg, semsc, semst):
    c = lax.axis_index("c")
    s = lax.axis_index("s")
    wid = s * 2 + c

    @pl.when(s == 0)
    def _():
        pltpu.sync_copy(zeros16, den_sh)

    plsc.subcore_barrier()

    row0 = wid * RPW
    pltpu.sync_copy(srcR.at[pl.ds(row0, RPW)], srcs_v)
    pltpu.sync_copy(dstR.at[pl.ds(row0, RPW)], dsts_v)

    def issue_gather(j, b):
        pltpu.async_copy(stab.at[srcs_v.at[j]], g1.at[b], semg.at[b])
        pltpu.async_copy(dtab.at[dsts_v.at[j]], g2.at[b], semg.at[b])

    issue_gather(0, 0)

    def row_group(g, carry):
        for b in range(2):
            j = 2 * g + b

            @pl.when(j + 1 < RPW)
            def _():
                issue_gather(j + 1, 1 - b)

            pltpu.make_async_copy(stab.at[srcs_v.at[j]], g1.at[b],
                                  semg.at[b]).wait()
            pltpu.make_async_copy(dtab.at[dsts_v.at[j]], g2.at[b],
                                  semg.at[b]).wait()

            @pl.when(j >= 2)
            def _():
                pltpu.make_async_copy(exv.at[b], den_sh.at[dsts_v.at[j]],
                                      semsc.at[b]).wait()
                pltpu.make_async_copy(exv.at[b], ex_out.at[row0 + j],
                                      semst.at[b]).wait()

            def e_body(e, carry2):
                a = g1[b, e, :] + g2[b, e, :]
                a = jnp.maximum(a, 0.2 * a)
                exv[b, e, :] = jnp.exp(a)
                return carry2

            lax.fori_loop(0, 128, e_body, 0, unroll=4)
            pltpu.async_copy(exv.at[b], den_sh.at[dsts_v.at[j]],
                             semsc.at[b], add=True)
            pltpu.async_copy(exv.at[b], ex_out.at[row0 + j], semst.at[b])
        return carry

    lax.fori_loop(0, RPW // 2, row_group, 0)
    for b in range(2):
        pltpu.make_async_copy(exv.at[b], den_sh.at[dsts_v.at[0]],
                              semsc.at[b]).wait()
        pltpu.make_async_copy(exv.at[b], ex_out.at[row0], semst.at[b]).wait()
    plsc.subcore_barrier()

    @pl.when(s == 0)
    def _():
        pltpu.sync_copy(den_sh, denp_out.at[c])


@functools.partial(
    pl.kernel,
    out_type=(
        jax.ShapeDtypeStruct((EE, HID), _f32),
        jax.ShapeDtypeStruct((2, NP, 64), _f32),
    ),
    mesh=_mesh,
    compiler_params=pltpu.CompilerParams(use_tc_tiling_on_sc=False),
    scratch_types=[
        pltpu.VMEM((RPW, 128), jnp.int32),
        pltpu.VMEM((RPW, 128), jnp.int32),
        pltpu.VMEM((2, 128, 64), _f32),
        pltpu.VMEM((2, 128, LANES), _f32),
        pltpu.VMEM((2, 128, LANES), _f32),
        pltpu.VMEM((2, 128, LANES), _f32),
        pltpu.VMEM((2, 128, LANES), _f32),
        pltpu.VMEM((2, 128, 64), _f32),
        pltpu.VMEM_SHARED((NP, 64), _f32),
        pltpu.SemaphoreType.DMA((2,)),
        pltpu.SemaphoreType.DMA((2,)),
        pltpu.SemaphoreType.DMA((2,)),
    ],
)
def _edge_agg(srcR, dstR, htab, den0, den1, exin, zeros64, aw_out, outp_out,
              srcs_v, dsts_v, hv, denv, denw, exv, awv, msgv, out_sh,
              semg, semsc, semst):
    c = lax.axis_index("c")
    s = lax.axis_index("s")
    wid = s * 2 + c

    @pl.when(s == 0)
    def _():
        pltpu.sync_copy(zeros64, out_sh)

    plsc.subcore_barrier()

    row0 = wid * RPW
    pltpu.sync_copy(srcR.at[pl.ds(row0, RPW)], srcs_v)
    pltpu.sync_copy(dstR.at[pl.ds(row0, RPW)], dsts_v)

    def issue_gather(j, b):
        pltpu.async_copy(htab.at[srcs_v.at[j]], hv.at[b], semg.at[b])
        pltpu.async_copy(den0.at[dsts_v.at[j]], denv.at[b], semg.at[b])
        pltpu.async_copy(den1.at[dsts_v.at[j]], denw.at[b], semg.at[b])
        pltpu.async_copy(exin.at[row0 + j], exv.at[b], semg.at[b])

    def aw_store(j, b):
        gr = row0 + j

        @pl.when(gr < FULLR)
        def _():
            pltpu.async_copy(awv.at[b, :, pl.ds(0, HID)],
                             aw_out.at[pl.ds(gr * 128, 128)], semst.at[b])

        @pl.when(gr == FULLR)
        def _():
            pltpu.async_copy(awv.at[b, pl.ds(0, TAIL), pl.ds(0, HID)],
                             aw_out.at[pl.ds(FULLR * 128, TAIL)], semst.at[b])

    def aw_wait(j, b):
        gr = row0 + j

        @pl.when(gr < FULLR)
        def _():
            pltpu.make_async_copy(awv.at[b, :, pl.ds(0, HID)],
                                  aw_out.at[pl.ds(0, 128)],
                                  semst.at[b]).wait()

        @pl.when(gr == FULLR)
        def _():
            pltpu.make_async_copy(awv.at[b, pl.ds(0, TAIL), pl.ds(0, HID)],
                                  aw_out.at[pl.ds(0, TAIL)],
                                  semst.at[b]).wait()

    issue_gather(0, 0)

    def row_group(g, carry):
        for b in range(2):
            j = 2 * g + b

            @pl.when(j + 1 < RPW)
            def _():
                issue_gather(j + 1, 1 - b)

            pltpu.make_async_copy(htab.at[srcs_v.at[j]], hv.at[b],
                                  semg.at[b]).wait()
            pltpu.make_async_copy(den0.at[dsts_v.at[j]], denv.at[b],
                                  semg.at[b]).wait()
            pltpu.make_async_copy(den1.at[dsts_v.at[j]], denw.at[b],
                                  semg.at[b]).wait()
            pltpu.make_async_copy(exin.at[row0 + j], exv.at[b],
                                  semg.at[b]).wait()

            @pl.when(j >= 2)
            def _():
                pltpu.make_async_copy(msgv.at[b], out_sh.at[dsts_v.at[j]],
                                      semsc.at[b]).wait()
                aw_wait(j - 2, b)

            lomask = lax.iota(jnp.int32, 16) < HID

            def e_body(e, carry2):
                alpha = exv[b, e, :] / (denv[b, e, :] + denw[b, e, :] + 1e-16)
                awv[b, e, :] = alpha
                for jj in range(4):
                    a_lo = jnp.broadcast_to(alpha[2 * jj], (16,))
                    a_hi = jnp.broadcast_to(alpha[2 * jj + 1], (16,))
                    rep = jnp.where(lomask, a_lo, a_hi)
                    msgv[b, e, pl.ds(jj * 16, 16)] = (
                        hv[b, e, pl.ds(jj * 16, 16)] * rep)
                return carry2

            lax.fori_loop(0, 128, e_body, 0, unroll=4)
            pltpu.async_copy(msgv.at[b], out_sh.at[dsts_v.at[j]],
                             semsc.at[b], add=True)
            aw_store(j, b)
        return carry

    lax.fori_loop(0, RPW // 2, row_group, 0)
    for b in range(2):
        pltpu.make_async_copy(msgv.at[b], out_sh.at[dsts_v.at[0]],
                              semsc.at[b]).wait()
        aw_wait(RPW - 2 + b, b)
    plsc.subcore_barrier()

    @pl.when(s == 0)
    def _():
        pltpu.sync_copy(out_sh, outp_out.at[c])


def _dense1_body(x_ref, w1_ref, wres_ref, s1m_ref, d1m_ref,
                 h1_ref, s1_ref, d1_ref, res_ref):
    xb = jnp.pad(x_ref[...], ((0, NP - N), (0, 0)))
    h1 = jnp.dot(xb, w1_ref[...], preferred_element_type=_f32)
    h1_ref[...] = h1
    s1_ref[...] = jnp.dot(h1, s1m_ref[...], preferred_element_type=_f32)
    d1_ref[...] = jnp.dot(h1, d1m_ref[...], preferred_element_type=_f32)
    res_ref[...] = jnp.dot(xb, wres_ref[...], preferred_element_type=_f32)


_dense1 = pl.pallas_call(
    _dense1_body,
    out_shape=[
        jax.ShapeDtypeStruct((NP, 64), _f32),
        jax.ShapeDtypeStruct((NP, LANES), _f32),
        jax.ShapeDtypeStruct((NP, LANES), _f32),
        jax.ShapeDtypeStruct((NP, 64), _f32),
    ],
)


def _prep_body(e_ref, ei_ref, srcR_ref, dstR_ref):
    erow = NUM_E_ROWS
    rid = lax.broadcasted_iota(jnp.int32, (ROWS, 128), 0)
    cid = lax.broadcasted_iota(jnp.int32, (ROWS, 128), 1)
    fid = rid * 128 + cid
    fill = jnp.where(fid < EE, fid - E, N)
    for k, out_ref in ((0, srcR_ref), (1, dstR_ref)):
        vals = e_ref[k].reshape(erow, 128)
        vals = jnp.pad(vals, ((0, ROWS - erow), (0, 0)))
        out_ref[...] = jnp.where(fid < E, vals, fill)
    lane = lax.broadcasted_iota(jnp.int32, (2, EE), 1)
    loop2 = lane - E
    ei_ref[...] = jnp.where(lane < E,
                            jnp.pad(e_ref[...], ((0, 0), (0, EE - E))),
                            loop2)


NUM_E_ROWS = E // 128
_prep = pl.pallas_call(
    _prep_body,
    out_shape=[
        jax.ShapeDtypeStruct((2, EE), jnp.int32),
        jax.ShapeDtypeStruct((ROWS, 128), jnp.int32),
        jax.ShapeDtypeStruct((ROWS, 128), jnp.int32),
    ],
)


def _mid_body(p_ref, res_ref, b1_ref, bres_ref, w2_ref, as2m_ref, ad2m_ref,
              h2t_ref, s2b_ref, d2b_ref):
    v = p_ref[0] + p_ref[1] + b1_ref[...] + bres_ref[...] + res_ref[...]
    hmid = jnp.where(v > 0, v, jnp.exp(v) - 1.0)
    h2t = jnp.dot(hmid, w2_ref[...], preferred_element_type=_f32)
    h2t_ref[...] = h2t
    s2b_ref[...] = jnp.dot(h2t, as2m_ref[...], preferred_element_type=_f32)
    d2b_ref[...] = jnp.dot(h2t, ad2m_ref[...], preferred_element_type=_f32)


_mid = pl.pallas_call(
    _mid_body,
    out_shape=[
        jax.ShapeDtypeStruct((NP, 64), _f32),
        jax.ShapeDtypeStruct((NP, LANES), _f32),
        jax.ShapeDtypeStruct((NP, LANES), _f32),
    ],
)


def _final_body(p_ref, b2_ref, o_ref):
    v = p_ref[0, :N] + p_ref[1, :N] + b2_ref[...]
    z = jnp.where(v > 0, v, jnp.exp(v) - 1.0)
    m = jnp.max(z, axis=1, keepdims=True)
    lse = jnp.log(jnp.sum(jnp.exp(z - m), axis=1, keepdims=True)) + m
    o_ref[...] = z - lse


_final = pl.pallas_call(
    _final_body,
    out_shape=jax.ShapeDtypeStruct((N, 64), _f32),
)


def kernel(x, edge_index, Wres, bres, W1, as1, ad1, b1, W2, as2, ad2, b2):
    ei, srcR, dstR = _prep(edge_index)

    eye8 = jnp.eye(H, dtype=_f32)
    s1m = jnp.pad((as1[0][:, :, None] * eye8[:, None, :]).reshape(H * HID, H),
                  ((0, 0), (0, LANES - H)))
    d1m = jnp.pad((ad1[0][:, :, None] * eye8[:, None, :]).reshape(H * HID, H),
                  ((0, 0), (0, LANES - H)))
    as2m = jnp.tile(as2[0, 0][:, None], (1, LANES))
    ad2m = jnp.tile(ad2[0, 0][:, None], (1, LANES))

    z16 = jnp.zeros((NP, LANES), _f32)
    z64 = jnp.zeros((NP, 64), _f32)

    h1, s1, d1, res = _dense1(x, W1, Wres, s1m, d1m)
    ex1, denp1 = _edge_num(srcR, dstR, s1, d1, z16)
    aw1e, outp1 = _edge_agg(srcR, dstR, h1, denp1[0], denp1[1], ex1, z64)
    h2t, s2b, d2b = _mid(outp1, res, b1.reshape(1, 64), bres.reshape(1, 64),
                         W2, as2m, ad2m)
    ex2, denp2 = _edge_num(srcR, dstR, s2b, d2b, z16)
    aw2e, outp2 = _edge_agg(srcR, dstR, h2t, denp2[0], denp2[1], ex2, z64)
    logp = _final(outp2, b2.reshape(1, C))

    return (logp, (ei, aw1e), (ei, aw2e[:, :1]))
